# Initial kernel scaffold; baseline (speedup 1.0000x reference)
import jax
import jax.numpy as jnp
from jax.experimental import pallas as pl


def kernel(x, edge_index, batch, params):
    raise NotImplementedError("write your pallas kernel here")


# speedup vs baseline: 6.0823x; 6.0823x over previous
import functools

import jax
import jax.numpy as jnp
from jax import lax
from jax.experimental import pallas as pl
from jax.experimental.pallas import tpu as pltpu
from jax.experimental.pallas import tpu_sc as plsc

N = 50000
E = 800000
IN_DIM = 12
HID = 64
NUM_LAYER = 5
NUM_CLASSES = 2
NUM_GRAPHS = 64

NUM_SC = 2
NUM_TEC = 16
HALF = N // NUM_SC
SUB = 80
NSUB = 5
CHUNK = SUB * NSUB
EDGES_PER_TEC = E // NUM_TEC
CHUNKS_PER_TEC = EDGES_PER_TEC // CHUNK
NCHUNK = E // CHUNK
ACC_ROWS = 25088
PER_TEC_ACC = ACC_ROWS // NUM_TEC
OUT_PER_TEC = 1560
OUT_REM = HALF - NUM_TEC * OUT_PER_TEC


@functools.lru_cache(maxsize=None)
def _make_seg_sum(d):
  mesh = plsc.VectorSubcoreMesh(core_axis_name="c", subcore_axis_name="s")

  @functools.partial(
      pl.kernel,
      mesh=mesh,
      compiler_params=pltpu.CompilerParams(use_tc_tiling_on_sc=False),
      out_type=jax.ShapeDtypeStruct((N, d), jnp.float32),
      scratch_types=[
          pltpu.VMEM((NSUB, SUB), jnp.int32),
          pltpu.VMEM((NSUB, SUB), jnp.int32),
          pltpu.VMEM((NSUB, SUB, d), jnp.float32),
          pltpu.VMEM_SHARED((ACC_ROWS, d), jnp.float32),
          pltpu.SemaphoreType.DMA,
      ],
  )
  def seg_sum(h_hbm, src2d_hbm, dst2d_hbm, out_hbm, src_v, dst_v, rows_v,
              acc, sem):
    core = lax.axis_index("c")
    sub = lax.axis_index("s")

    def zero_body(i, carry):
      for j in range(NSUB):
        for cc in range(d // 16):
          rows_v[j, i, pl.ds(cc * 16, 16)] = jnp.zeros((16,), jnp.float32)
      return carry
    lax.fori_loop(0, SUB, zero_body, 0)

    base_acc = sub * PER_TEC_ACC
    n_zc = PER_TEC_ACC // SUB
    def zcopy(m, carry):
      pltpu.sync_copy(rows_v.at[0], acc.at[pl.ds(base_acc + m * SUB, SUB)])
      return carry
    lax.fori_loop(0, n_zc, zcopy, 0)
    ztail = PER_TEC_ACC - n_zc * SUB
    if ztail:
      pltpu.sync_copy(rows_v.at[0, pl.ds(0, ztail)],
                      acc.at[pl.ds(base_acc + n_zc * SUB, ztail)])

    plsc.subcore_barrier()

    g0 = sub * CHUNKS_PER_TEC
    dst_g0 = core * NCHUNK + g0
    def chunk_body(k, carry):
      pltpu.sync_copy(src2d_hbm.at[g0 + k], src_v)
      pltpu.sync_copy(dst2d_hbm.at[dst_g0 + k], dst_v)
      descs = [pltpu.async_copy(h_hbm.at[src_v.at[j]], rows_v.at[j], sem)
               for j in range(NSUB)]
      for dsc in descs:
        dsc.wait()
      for j in range(NSUB):
        pltpu.sync_copy(rows_v.at[j], acc.at[dst_v.at[j]], add=True)
      return carry
    lax.fori_loop(0, CHUNKS_PER_TEC, chunk_body, 0)

    plsc.subcore_barrier()

    out_l = sub * OUT_PER_TEC
    out_g = core * HALF + sub * OUT_PER_TEC
    n_ec = OUT_PER_TEC // SUB
    def ecopy(m, carry):
      pltpu.sync_copy(acc.at[pl.ds(out_l + m * SUB, SUB)], rows_v.at[0])
      pltpu.sync_copy(rows_v.at[0], out_hbm.at[pl.ds(out_g + m * SUB, SUB)])
      return carry
    lax.fori_loop(0, n_ec, ecopy, 0)
    etail = OUT_PER_TEC - n_ec * SUB
    if etail:
      pltpu.sync_copy(acc.at[pl.ds(out_l + n_ec * SUB, etail)],
                      rows_v.at[0, pl.ds(0, etail)])
      pltpu.sync_copy(rows_v.at[0, pl.ds(0, etail)],
                      out_hbm.at[pl.ds(out_g + n_ec * SUB, etail)])

    @pl.when(sub == NUM_TEC - 1)
    def _():
      pltpu.sync_copy(acc.at[pl.ds(NUM_TEC * OUT_PER_TEC, OUT_REM)],
                      rows_v.at[1, pl.ds(0, OUT_REM)])
      pltpu.sync_copy(rows_v.at[1, pl.ds(0, OUT_REM)],
                      out_hbm.at[pl.ds(core * HALF + NUM_TEC * OUT_PER_TEC,
                                       OUT_REM)])

  return seg_sum


BLK = 1000


def _mlp_body(eps_ref, h_ref, agg_ref, w1_ref, b1_ref, w2_ref, b2_ref,
              out_ref):
  z = (1.0 + eps_ref[0]) * h_ref[...] + agg_ref[...]
  z = jnp.maximum(
      jnp.dot(z, w1_ref[...], preferred_element_type=jnp.float32)
      + b1_ref[...], 0.0)
  z = jnp.dot(z, w2_ref[...], preferred_element_type=jnp.float32) + b2_ref[...]
  out_ref[...] = jnp.maximum(z, 0.0)


def _mlp(eps, h, agg, w1, b1, w2, b2):
  din = h.shape[1]
  return pl.pallas_call(
      _mlp_body,
      grid=(N // BLK,),
      in_specs=[
          pl.BlockSpec(memory_space=pltpu.SMEM),
          pl.BlockSpec((BLK, din), lambda i: (i, 0)),
          pl.BlockSpec((BLK, din), lambda i: (i, 0)),
          pl.BlockSpec((din, HID), lambda i: (0, 0)),
          pl.BlockSpec((1, HID), lambda i: (0, 0)),
          pl.BlockSpec((HID, HID), lambda i: (0, 0)),
          pl.BlockSpec((1, HID), lambda i: (0, 0)),
      ],
      out_specs=pl.BlockSpec((BLK, HID), lambda i: (i, 0)),
      out_shape=jax.ShapeDtypeStruct((N, HID), jnp.float32),
  )(eps.reshape(1), h, agg, w1, b1, w2, b2)


def _pool_body(h_ref, batch_ref, wc_ref, bc_ref, out_ref, sums_ref, cnts_ref):
  i = pl.program_id(0)

  @pl.when(i == 0)
  def _():
    sums_ref[...] = jnp.zeros_like(sums_ref)
    cnts_ref[...] = jnp.zeros_like(cnts_ref)

  b = batch_ref[0, 0, :]
  oh = (b[:, None] == lax.broadcasted_iota(jnp.int32, (BLK, NUM_GRAPHS), 1)
        ).astype(jnp.float32)
  sums_ref[...] += lax.dot_general(oh, h_ref[...], (((0,), (0,)), ((), ())),
                                   preferred_element_type=jnp.float32)
  cnts_ref[...] += lax.dot_general(oh, oh, (((0,), (0,)), ((), ())),
                                   preferred_element_type=jnp.float32)

  @pl.when(i == pl.num_programs(0) - 1)
  def _():
    cb = jnp.dot(cnts_ref[...], jnp.ones((NUM_GRAPHS, NUM_GRAPHS),
                                         jnp.float32),
                 preferred_element_type=jnp.float32)
    pooled = sums_ref[...] / jnp.maximum(cb, 1.0)
    out_ref[...] = jnp.dot(pooled, wc_ref[...],
                           preferred_element_type=jnp.float32) + bc_ref[...]


def _pool(h, batch3d, wc, bc):
  return pl.pallas_call(
      _pool_body,
      grid=(N // BLK,),
      in_specs=[
          pl.BlockSpec((BLK, HID), lambda i: (i, 0)),
          pl.BlockSpec((1, 1, BLK), lambda i: (i, 0, 0)),
          pl.BlockSpec((HID, NUM_CLASSES), lambda i: (0, 0)),
          pl.BlockSpec((1, NUM_CLASSES), lambda i: (0, 0)),
      ],
      out_specs=pl.BlockSpec((NUM_GRAPHS, NUM_CLASSES), lambda i: (0, 0)),
      out_shape=jax.ShapeDtypeStruct((NUM_GRAPHS, NUM_CLASSES), jnp.float32),
      scratch_shapes=[
          pltpu.VMEM((NUM_GRAPHS, HID), jnp.float32),
          pltpu.VMEM((NUM_GRAPHS, NUM_GRAPHS), jnp.float32),
      ],
  )(h, batch3d, wc, bc)


def kernel(x, edge_index, batch, params):
  src = edge_index[0]
  dst = edge_index[1]
  src2d = src.reshape(NCHUNK, NSUB, SUB)

  loc = []
  for c in range(NUM_SC):
    lcl = dst - c * HALF
    oob = (lcl < 0) | (lcl >= HALF)
    lcl = jnp.where(oob, HALF + (dst & 63), lcl)
    loc.append(lcl.reshape(NCHUNK, NSUB, SUB))
  dst2d = jnp.concatenate(loc, axis=0)

  x16 = jnp.pad(x, ((0, 0), (0, 16 - IN_DIM)))
  w1_0 = jnp.pad(params["W1"][0], ((0, 16 - IN_DIM), (0, 0)))
  batch3d = batch.reshape(N // BLK, 1, BLK)

  h = x16
  for layer in range(NUM_LAYER):
    agg = _make_seg_sum(16 if layer == 0 else HID)(h, src2d, dst2d)
    w1 = w1_0 if layer == 0 else params["W1"][layer]
    h = _mlp(params["eps"][layer], h, agg, w1,
             params["b1"][layer].reshape(1, HID),
             params["W2"][layer], params["b2"][layer].reshape(1, HID))
  return _pool(h, batch3d, params["Wc"], params["bc"].reshape(1, NUM_CLASSES))


# speedup vs baseline: 6.7479x; 1.1094x over previous
import functools

import jax
import jax.numpy as jnp
from jax import lax
from jax.experimental import pallas as pl
from jax.experimental.pallas import tpu as pltpu
from jax.experimental.pallas import tpu_sc as plsc

N = 50000
E = 800000
IN_DIM = 12
HID = 64
NUM_LAYER = 5
NUM_CLASSES = 2
NUM_GRAPHS = 64

NUM_SC = 2
NUM_TEC = 16
SUB = 80
NSUB = 5
CHUNK = SUB * NSUB
EDGES_PER_TEC = E // NUM_TEC
CHUNKS_PER_TEC = EDGES_PER_TEC // CHUNK
NCHUNK = E // CHUNK
PER_TEC_ACC = N // NUM_TEC


def _make_seg_sum(d2):
  mesh = plsc.VectorSubcoreMesh(core_axis_name="c", subcore_axis_name="s")

  @functools.partial(
      pl.kernel,
      mesh=mesh,
      compiler_params=pltpu.CompilerParams(use_tc_tiling_on_sc=False),
      out_type=jax.ShapeDtypeStruct((NUM_SC, N, d2), jnp.float32),
      scratch_types=[
          pltpu.VMEM((NSUB, SUB), jnp.int32),
          pltpu.VMEM((NSUB, SUB), jnp.int32),
          pltpu.VMEM((NSUB, SUB, d2), jnp.float32),
          pltpu.VMEM_SHARED((N, d2), jnp.float32),
          pltpu.SemaphoreType.DMA,
      ],
  )
  def seg_sum(h_hbm, src2d_hbm, dst2d_hbm, out_hbm, src_v, dst_v, rows_v,
              acc, sem):
    core = lax.axis_index("c")
    sub = lax.axis_index("s")
    h_half = h_hbm.at[core]
    out_half = out_hbm.at[core]

    def zero_body(i, carry):
      for j in range(NSUB):
        for cc in range(d2 // 16):
          rows_v[j, i, pl.ds(cc * 16, 16)] = jnp.zeros((16,), jnp.float32)
      return carry
    lax.fori_loop(0, SUB, zero_body, 0)

    base_acc = sub * PER_TEC_ACC
    n_zc = PER_TEC_ACC // SUB
    def zcopy(m, carry):
      pltpu.sync_copy(rows_v.at[0], acc.at[pl.ds(base_acc + m * SUB, SUB)])
      return carry
    lax.fori_loop(0, n_zc, zcopy, 0)
    ztail = PER_TEC_ACC - n_zc * SUB
    if ztail:
      pltpu.sync_copy(rows_v.at[0, pl.ds(0, ztail)],
                      acc.at[pl.ds(base_acc + n_zc * SUB, ztail)])

    plsc.subcore_barrier()

    g0 = sub * CHUNKS_PER_TEC
    def chunk_body(k, carry):
      pltpu.sync_copy(src2d_hbm.at[g0 + k], src_v)
      pltpu.sync_copy(dst2d_hbm.at[g0 + k], dst_v)
      descs = [pltpu.async_copy(h_half.at[src_v.at[j]], rows_v.at[j], sem)
               for j in range(NSUB)]
      for dsc in descs:
        dsc.wait()
      for j in range(NSUB):
        pltpu.sync_copy(rows_v.at[j], acc.at[dst_v.at[j]], add=True)
      return carry
    lax.fori_loop(0, CHUNKS_PER_TEC, chunk_body, 0)

    plsc.subcore_barrier()

    out_l = sub * PER_TEC_ACC
    n_ec = PER_TEC_ACC // SUB
    def ecopy(m, carry):
      pltpu.sync_copy(acc.at[pl.ds(out_l + m * SUB, SUB)], rows_v.at[0])
      pltpu.sync_copy(rows_v.at[0], out_half.at[pl.ds(out_l + m * SUB, SUB)])
      return carry
    lax.fori_loop(0, n_ec, ecopy, 0)
    etail = PER_TEC_ACC - n_ec * SUB
    if etail:
      pltpu.sync_copy(acc.at[pl.ds(out_l + n_ec * SUB, etail)],
                      rows_v.at[1, pl.ds(0, etail)])
      pltpu.sync_copy(rows_v.at[1, pl.ds(0, etail)],
                      out_half.at[pl.ds(out_l + n_ec * SUB, etail)])

  return seg_sum


_SEG = {}


def _seg_sum(d2):
  if d2 not in _SEG:
    _SEG[d2] = _make_seg_sum(d2)
  return _SEG[d2]


BLK = 1000


def _mlp_body(eps_ref, h_ref, agg_ref, w1_ref, b1_ref, w2_ref, b2_ref,
              out_ref):
  h = jnp.concatenate([h_ref[0], h_ref[1]], axis=1)
  agg = jnp.concatenate([agg_ref[0], agg_ref[1]], axis=1)
  z = (1.0 + eps_ref[0]) * h + agg
  z = jnp.maximum(
      jnp.dot(z, w1_ref[...], preferred_element_type=jnp.float32)
      + b1_ref[...], 0.0)
  z = jnp.dot(z, w2_ref[...], preferred_element_type=jnp.float32) + b2_ref[...]
  z = jnp.maximum(z, 0.0)
  out_ref[0] = z[:, :HID // 2]
  out_ref[1] = z[:, HID // 2:]


def _mlp(eps, h3, agg3, w1, b1, w2, b2):
  d2 = h3.shape[2]
  return pl.pallas_call(
      _mlp_body,
      grid=(N // BLK,),
      in_specs=[
          pl.BlockSpec(memory_space=pltpu.SMEM),
          pl.BlockSpec((NUM_SC, BLK, d2), lambda i: (0, i, 0)),
          pl.BlockSpec((NUM_SC, BLK, d2), lambda i: (0, i, 0)),
          pl.BlockSpec((2 * d2, HID), lambda i: (0, 0)),
          pl.BlockSpec((1, HID), lambda i: (0, 0)),
          pl.BlockSpec((HID, HID), lambda i: (0, 0)),
          pl.BlockSpec((1, HID), lambda i: (0, 0)),
      ],
      out_specs=pl.BlockSpec((NUM_SC, BLK, HID // 2), lambda i: (0, i, 0)),
      out_shape=jax.ShapeDtypeStruct((NUM_SC, N, HID // 2), jnp.float32),
  )(eps.reshape(1), h3, agg3, w1, b1, w2, b2)


def _pool_body(h_ref, batch_ref, wc_ref, bc_ref, out_ref, sums_ref, cnts_ref):
  i = pl.program_id(0)

  @pl.when(i == 0)
  def _():
    sums_ref[...] = jnp.zeros_like(sums_ref)
    cnts_ref[...] = jnp.zeros_like(cnts_ref)

  h = jnp.concatenate([h_ref[0], h_ref[1]], axis=1)
  b = batch_ref[0, 0, :]
  oh = (b[:, None] == lax.broadcasted_iota(jnp.int32, (BLK, NUM_GRAPHS), 1)
        ).astype(jnp.float32)
  sums_ref[...] += lax.dot_general(oh, h, (((0,), (0,)), ((), ())),
                                   preferred_element_type=jnp.float32)
  cnts_ref[...] += lax.dot_general(oh, oh, (((0,), (0,)), ((), ())),
                                   preferred_element_type=jnp.float32)

  @pl.when(i == pl.num_programs(0) - 1)
  def _():
    cb = jnp.dot(cnts_ref[...], jnp.ones((NUM_GRAPHS, NUM_GRAPHS),
                                         jnp.float32),
                 preferred_element_type=jnp.float32)
    pooled = sums_ref[...] / jnp.maximum(cb, 1.0)
    out_ref[...] = jnp.dot(pooled, wc_ref[...],
                           preferred_element_type=jnp.float32) + bc_ref[...]


def _pool(h3, batch3d, wc, bc):
  return pl.pallas_call(
      _pool_body,
      grid=(N // BLK,),
      in_specs=[
          pl.BlockSpec((NUM_SC, BLK, HID // 2), lambda i: (0, i, 0)),
          pl.BlockSpec((1, 1, BLK), lambda i: (i, 0, 0)),
          pl.BlockSpec((HID, NUM_CLASSES), lambda i: (0, 0)),
          pl.BlockSpec((1, NUM_CLASSES), lambda i: (0, 0)),
      ],
      out_specs=pl.BlockSpec((NUM_GRAPHS, NUM_CLASSES), lambda i: (0, 0)),
      out_shape=jax.ShapeDtypeStruct((NUM_GRAPHS, NUM_CLASSES), jnp.float32),
      scratch_shapes=[
          pltpu.VMEM((NUM_GRAPHS, HID), jnp.float32),
          pltpu.VMEM((NUM_GRAPHS, NUM_GRAPHS), jnp.float32),
      ],
  )(h3, batch3d, wc, bc)


def kernel(x, edge_index, batch, params):
  src2d = edge_index[0].reshape(NCHUNK, NSUB, SUB)
  dst2d = edge_index[1].reshape(NCHUNK, NSUB, SUB)

  x32 = jnp.pad(x, ((0, 0), (0, 32 - IN_DIM)))
  h3 = x32.reshape(N, NUM_SC, 16).transpose(1, 0, 2)
  w1_0 = jnp.pad(params["W1"][0], ((0, 32 - IN_DIM), (0, 0)))
  batch3d = batch.reshape(N // BLK, 1, BLK)

  for layer in range(NUM_LAYER):
    d2 = h3.shape[2]
    agg3 = _seg_sum(d2)(h3, src2d, dst2d)
    w1 = w1_0 if layer == 0 else params["W1"][layer]
    h3 = _mlp(params["eps"][layer], h3, agg3, w1,
              params["b1"][layer].reshape(1, HID),
              params["W2"][layer], params["b2"][layer].reshape(1, HID))
  return _pool(h3, batch3d, params["Wc"], params["bc"].reshape(1, NUM_CLASSES))


# speedup vs baseline: 12.0581x; 1.7869x over previous
import functools

import jax
import jax.numpy as jnp
from jax import lax
from jax.experimental import pallas as pl
from jax.experimental.pallas import tpu as pltpu
from jax.experimental.pallas import tpu_sc as plsc

N = 50000
E = 800000
IN_DIM = 12
HID = 64
NUM_LAYER = 5
NUM_CLASSES = 2
NUM_GRAPHS = 64

NUM_SC = 2
NUM_TEC = 16
SUB = 80
NSUB = 5
CHUNK = SUB * NSUB
EDGES_PER_TEC = E // NUM_TEC
CHUNKS_PER_TEC = EDGES_PER_TEC // CHUNK
NCHUNK = E // CHUNK
PER_TEC_ACC = N // NUM_TEC


def _make_seg_sum(d2):
  mesh = plsc.VectorSubcoreMesh(core_axis_name="c", subcore_axis_name="s")

  @functools.partial(
      pl.kernel,
      mesh=mesh,
      compiler_params=pltpu.CompilerParams(use_tc_tiling_on_sc=False),
      out_type=jax.ShapeDtypeStruct((NUM_SC, N, d2), jnp.float32),
      scratch_types=[
          pltpu.VMEM((3, 2, NSUB, SUB), jnp.int32),
          pltpu.VMEM((2, NSUB, SUB, d2), jnp.float32),
          pltpu.VMEM_SHARED((N, d2), jnp.float32),
          pltpu.SemaphoreType.DMA,
          pltpu.SemaphoreType.DMA,
          pltpu.SemaphoreType.DMA,
      ],
  )
  def seg_sum(h_hbm, idx_hbm, out_hbm, idx_v, rows_v,
              acc, sem_i, sem_g, sem_s):
    core = lax.axis_index("c")
    sub = lax.axis_index("s")
    h_half = h_hbm.at[core]
    out_half = out_hbm.at[core]

    def zero_body(i, carry):
      for cc in range(d2 // 16):
        rows_v[0, 0, i, pl.ds(cc * 16, 16)] = jnp.zeros((16,), jnp.float32)
      return carry
    lax.fori_loop(0, SUB, zero_body, 0)

    base_acc = sub * PER_TEC_ACC
    n_zc = PER_TEC_ACC // SUB
    def zcopy(m, carry):
      pltpu.sync_copy(rows_v.at[0, 0], acc.at[pl.ds(base_acc + m * SUB, SUB)])
      return carry
    lax.fori_loop(0, n_zc, zcopy, 0)
    ztail = PER_TEC_ACC - n_zc * SUB
    if ztail:
      pltpu.sync_copy(rows_v.at[0, 0, pl.ds(0, ztail)],
                      acc.at[pl.ds(base_acc + n_zc * SUB, ztail)])

    plsc.subcore_barrier()

    g0 = sub * CHUNKS_PER_TEC
    nc = CHUNKS_PER_TEC

    def _wait_rows(sem, slot):
      for j in range(NSUB):
        pltpu.make_async_copy(h_half.at[pl.ds(0, SUB)], rows_v.at[slot, j],
                              sem).wait()

    pltpu.async_copy(idx_hbm.at[g0], idx_v.at[0], sem_i)

    def pipe_body(k, carry):
      s2 = lax.rem(k, 2)
      s3 = lax.rem(k, 3)
      p2 = lax.rem(k + 1, 2)
      m3 = lax.rem(k + 2, 3)

      @pl.when(k >= 2)
      def _():
        _wait_rows(sem_s, s2)

      @pl.when(k < nc)
      def _():
        pltpu.make_async_copy(idx_hbm.at[g0 + k], idx_v.at[s3], sem_i).wait()
        for j in range(NSUB):
          pltpu.async_copy(h_half.at[idx_v.at[s3, 0, j]], rows_v.at[s2, j],
                           sem_g)

      @pl.when(k + 1 < nc)
      def _():
        pltpu.async_copy(idx_hbm.at[g0 + k + 1], idx_v.at[lax.rem(k + 1, 3)],
                         sem_i)

      @pl.when((k >= 1) & (k <= nc))
      def _():
        _wait_rows(sem_g, p2)
        for j in range(NSUB):
          pltpu.async_copy(rows_v.at[p2, j], acc.at[idx_v.at[m3, 1, j]],
                           sem_s, add=True)
      return carry

    lax.fori_loop(0, nc + 2, pipe_body, 0)

    plsc.subcore_barrier()

    out_l = sub * PER_TEC_ACC
    n_ec = PER_TEC_ACC // SUB
    def ecopy(m, carry):
      pltpu.sync_copy(acc.at[pl.ds(out_l + m * SUB, SUB)], rows_v.at[0, 0])
      pltpu.sync_copy(rows_v.at[0, 0],
                      out_half.at[pl.ds(out_l + m * SUB, SUB)])
      return carry
    lax.fori_loop(0, n_ec, ecopy, 0)
    etail = PER_TEC_ACC - n_ec * SUB
    if etail:
      pltpu.sync_copy(acc.at[pl.ds(out_l + n_ec * SUB, etail)],
                      rows_v.at[0, 1, pl.ds(0, etail)])
      pltpu.sync_copy(rows_v.at[0, 1, pl.ds(0, etail)],
                      out_half.at[pl.ds(out_l + n_ec * SUB, etail)])

  return seg_sum


_SEG = {}


def _seg_sum(d2):
  if d2 not in _SEG:
    _SEG[d2] = _make_seg_sum(d2)
  return _SEG[d2]


BLK = 1000


def _mlp_body(eps_ref, h_ref, agg_ref, w1_ref, b1_ref, w2_ref, b2_ref,
              out_ref):
  h = jnp.concatenate([h_ref[0], h_ref[1]], axis=1)
  agg = jnp.concatenate([agg_ref[0], agg_ref[1]], axis=1)
  z = (1.0 + eps_ref[0]) * h + agg
  z = jnp.maximum(
      jnp.dot(z, w1_ref[...], preferred_element_type=jnp.float32)
      + b1_ref[...], 0.0)
  z = jnp.dot(z, w2_ref[...], preferred_element_type=jnp.float32) + b2_ref[...]
  z = jnp.maximum(z, 0.0)
  out_ref[0] = z[:, :HID // 2]
  out_ref[1] = z[:, HID // 2:]


def _mlp(eps, h3, agg3, w1, b1, w2, b2):
  d2 = h3.shape[2]
  return pl.pallas_call(
      _mlp_body,
      grid=(N // BLK,),
      in_specs=[
          pl.BlockSpec(memory_space=pltpu.SMEM),
          pl.BlockSpec((NUM_SC, BLK, d2), lambda i: (0, i, 0)),
          pl.BlockSpec((NUM_SC, BLK, d2), lambda i: (0, i, 0)),
          pl.BlockSpec((2 * d2, HID), lambda i: (0, 0)),
          pl.BlockSpec((1, HID), lambda i: (0, 0)),
          pl.BlockSpec((HID, HID), lambda i: (0, 0)),
          pl.BlockSpec((1, HID), lambda i: (0, 0)),
      ],
      out_specs=pl.BlockSpec((NUM_SC, BLK, HID // 2), lambda i: (0, i, 0)),
      out_shape=jax.ShapeDtypeStruct((NUM_SC, N, HID // 2), jnp.float32),
  )(eps.reshape(1), h3, agg3, w1, b1, w2, b2)


def _pool_body(h_ref, batch_ref, wc_ref, bc_ref, out_ref, sums_ref, cnts_ref):
  i = pl.program_id(0)

  @pl.when(i == 0)
  def _():
    sums_ref[...] = jnp.zeros_like(sums_ref)
    cnts_ref[...] = jnp.zeros_like(cnts_ref)

  h = jnp.concatenate([h_ref[0], h_ref[1]], axis=1)
  b = batch_ref[0, 0, :]
  oh = (b[:, None] == lax.broadcasted_iota(jnp.int32, (BLK, NUM_GRAPHS), 1)
        ).astype(jnp.float32)
  sums_ref[...] += lax.dot_general(oh, h, (((0,), (0,)), ((), ())),
                                   preferred_element_type=jnp.float32)
  cnts_ref[...] += lax.dot_general(oh, oh, (((0,), (0,)), ((), ())),
                                   preferred_element_type=jnp.float32)

  @pl.when(i == pl.num_programs(0) - 1)
  def _():
    cb = jnp.dot(cnts_ref[...], jnp.ones((NUM_GRAPHS, NUM_GRAPHS),
                                         jnp.float32),
                 preferred_element_type=jnp.float32)
    pooled = sums_ref[...] / jnp.maximum(cb, 1.0)
    out_ref[...] = jnp.dot(pooled, wc_ref[...],
                           preferred_element_type=jnp.float32) + bc_ref[...]


def _pool(h3, batch3d, wc, bc):
  return pl.pallas_call(
      _pool_body,
      grid=(N // BLK,),
      in_specs=[
          pl.BlockSpec((NUM_SC, BLK, HID // 2), lambda i: (0, i, 0)),
          pl.BlockSpec((1, 1, BLK), lambda i: (i, 0, 0)),
          pl.BlockSpec((HID, NUM_CLASSES), lambda i: (0, 0)),
          pl.BlockSpec((1, NUM_CLASSES), lambda i: (0, 0)),
      ],
      out_specs=pl.BlockSpec((NUM_GRAPHS, NUM_CLASSES), lambda i: (0, 0)),
      out_shape=jax.ShapeDtypeStruct((NUM_GRAPHS, NUM_CLASSES), jnp.float32),
      scratch_shapes=[
          pltpu.VMEM((NUM_GRAPHS, HID), jnp.float32),
          pltpu.VMEM((NUM_GRAPHS, NUM_GRAPHS), jnp.float32),
      ],
  )(h3, batch3d, wc, bc)


def kernel(x, edge_index, batch, params):
  idx_all = jnp.stack([edge_index[0].reshape(NCHUNK, NSUB, SUB),
                       edge_index[1].reshape(NCHUNK, NSUB, SUB)], axis=1)

  x32 = jnp.pad(x, ((0, 0), (0, 32 - IN_DIM)))
  h3 = x32.reshape(N, NUM_SC, 16).transpose(1, 0, 2)
  w1_0 = jnp.pad(params["W1"][0], ((0, 32 - IN_DIM), (0, 0)))
  batch3d = batch.reshape(N // BLK, 1, BLK)

  for layer in range(NUM_LAYER):
    d2 = h3.shape[2]
    agg3 = _seg_sum(d2)(h3, idx_all)
    w1 = w1_0 if layer == 0 else params["W1"][layer]
    h3 = _mlp(params["eps"][layer], h3, agg3, w1,
              params["b1"][layer].reshape(1, HID),
              params["W2"][layer], params["b2"][layer].reshape(1, HID))
  return _pool(h3, batch3d, params["Wc"], params["bc"].reshape(1, NUM_CLASSES))


# speedup vs baseline: 15.1016x; 1.2524x over previous
import functools

import jax
import jax.numpy as jnp
from jax import lax
from jax.experimental import pallas as pl
from jax.experimental.pallas import tpu as pltpu
from jax.experimental.pallas import tpu_sc as plsc

N = 50000
E = 800000
IN_DIM = 12
HID = 64
NUM_LAYER = 5
NUM_CLASSES = 2
NUM_GRAPHS = 64

NUM_SC = 2
NUM_TEC = 16
SUB = 80
NSUB = 5
CHUNK = SUB * NSUB
EDGES_PER_TEC = E // NUM_TEC
CHUNKS_PER_TEC = EDGES_PER_TEC // CHUNK
NCHUNK = E // CHUNK
NQ = 4
NR = 12544
NPAD = NQ * NR
BLKR = 256
NGRID = NR // BLKR
PER_TEC_ACC = NPAD // NUM_TEC


def _make_seg_sum(d2):
  mesh = plsc.VectorSubcoreMesh(core_axis_name="c", subcore_axis_name="s")

  @functools.partial(
      pl.kernel,
      mesh=mesh,
      compiler_params=pltpu.CompilerParams(use_tc_tiling_on_sc=False),
      out_type=jax.ShapeDtypeStruct((NUM_SC, NPAD, d2), jnp.float32),
      scratch_types=[
          pltpu.VMEM((3, 2, NSUB, SUB), jnp.int32),
          pltpu.VMEM((2, NSUB, SUB, d2), jnp.float32),
          pltpu.VMEM_SHARED((NPAD, d2), jnp.float32),
          pltpu.SemaphoreType.DMA,
          pltpu.SemaphoreType.DMA,
          pltpu.SemaphoreType.DMA,
      ],
  )
  def seg_sum(h_hbm, idx_hbm, out_hbm, idx_v, rows_v,
              acc, sem_i, sem_g, sem_s):
    core = lax.axis_index("c")
    sub = lax.axis_index("s")
    h_half = h_hbm.at[core]
    out_half = out_hbm.at[core]

    def zero_body(i, carry):
      for cc in range(d2 // 16):
        rows_v[0, 0, i, pl.ds(cc * 16, 16)] = jnp.zeros((16,), jnp.float32)
      return carry
    lax.fori_loop(0, SUB, zero_body, 0)

    base_acc = sub * PER_TEC_ACC
    n_zc = PER_TEC_ACC // SUB
    def zcopy(m, carry):
      pltpu.sync_copy(rows_v.at[0, 0], acc.at[pl.ds(base_acc + m * SUB, SUB)])
      return carry
    lax.fori_loop(0, n_zc, zcopy, 0)
    ztail = PER_TEC_ACC - n_zc * SUB
    if ztail:
      pltpu.sync_copy(rows_v.at[0, 0, pl.ds(0, ztail)],
                      acc.at[pl.ds(base_acc + n_zc * SUB, ztail)])

    plsc.subcore_barrier()

    g0 = sub * CHUNKS_PER_TEC
    nc = CHUNKS_PER_TEC

    def _wait_rows(sem, slot):
      for j in range(NSUB):
        pltpu.make_async_copy(h_half.at[pl.ds(0, SUB)], rows_v.at[slot, j],
                              sem).wait()

    pltpu.async_copy(idx_hbm.at[g0], idx_v.at[0], sem_i)

    def pipe_body(k, carry):
      s2 = lax.rem(k, 2)
      s3 = lax.rem(k, 3)
      p2 = lax.rem(k + 1, 2)
      m3 = lax.rem(k + 2, 3)

      @pl.when(k >= 2)
      def _():
        _wait_rows(sem_s, s2)

      @pl.when(k < nc)
      def _():
        pltpu.make_async_copy(idx_hbm.at[g0 + k], idx_v.at[s3], sem_i).wait()
        for j in range(NSUB):
          pltpu.async_copy(h_half.at[idx_v.at[s3, 0, j]], rows_v.at[s2, j],
                           sem_g)

      @pl.when(k + 1 < nc)
      def _():
        pltpu.async_copy(idx_hbm.at[g0 + k + 1], idx_v.at[lax.rem(k + 1, 3)],
                         sem_i)

      @pl.when((k >= 1) & (k <= nc))
      def _():
        _wait_rows(sem_g, p2)
        for j in range(NSUB):
          pltpu.async_copy(rows_v.at[p2, j], acc.at[idx_v.at[m3, 1, j]],
                           sem_s, add=True)
      return carry

    lax.fori_loop(0, nc + 2, pipe_body, 0)

    plsc.subcore_barrier()

    out_l = sub * PER_TEC_ACC
    n_ec = PER_TEC_ACC // SUB
    def ecopy(m, carry):
      pltpu.sync_copy(acc.at[pl.ds(out_l + m * SUB, SUB)], rows_v.at[0, 0])
      pltpu.sync_copy(rows_v.at[0, 0],
                      out_half.at[pl.ds(out_l + m * SUB, SUB)])
      return carry
    lax.fori_loop(0, n_ec, ecopy, 0)
    etail = PER_TEC_ACC - n_ec * SUB
    if etail:
      pltpu.sync_copy(acc.at[pl.ds(out_l + n_ec * SUB, etail)],
                      rows_v.at[0, 1, pl.ds(0, etail)])
      pltpu.sync_copy(rows_v.at[0, 1, pl.ds(0, etail)],
                      out_half.at[pl.ds(out_l + n_ec * SUB, etail)])

  return seg_sum


_SEG = {}


def _seg_sum(d2):
  if d2 not in _SEG:
    _SEG[d2] = _make_seg_sum(d2)
  return _SEG[d2]


def _mlp4_body(eps_ref, h_ref, agg_ref, w1_ref, b1_ref, w2_ref, b2_ref,
               out_ref):
  e = 1.0 + eps_ref[0]
  z0 = e * h_ref[0] + agg_ref[0]
  z1 = e * h_ref[1] + agg_ref[1]
  cols = [[], []]
  for q in range(NQ):
    zq = jnp.concatenate([z0[:, 32 * q:32 * q + 32],
                          z1[:, 32 * q:32 * q + 32]], axis=1)
    zq = jnp.maximum(
        jnp.dot(zq, w1_ref[...], preferred_element_type=jnp.float32)
        + b1_ref[...], 0.0)
    zq = jnp.dot(zq, w2_ref[...], preferred_element_type=jnp.float32)
    zq = jnp.maximum(zq + b2_ref[...], 0.0)
    cols[0].append(zq[:, :HID // 2])
    cols[1].append(zq[:, HID // 2:])
  out_ref[0] = jnp.concatenate(cols[0], axis=1)
  out_ref[1] = jnp.concatenate(cols[1], axis=1)


def _mlp4(eps, h4, agg4, w1, b1, w2, b2):
  return pl.pallas_call(
      _mlp4_body,
      grid=(NGRID,),
      in_specs=[
          pl.BlockSpec(memory_space=pltpu.SMEM),
          pl.BlockSpec((NUM_SC, BLKR, 128), lambda i: (0, i, 0)),
          pl.BlockSpec((NUM_SC, BLKR, 128), lambda i: (0, i, 0)),
          pl.BlockSpec((HID, HID), lambda i: (0, 0)),
          pl.BlockSpec((1, HID), lambda i: (0, 0)),
          pl.BlockSpec((HID, HID), lambda i: (0, 0)),
          pl.BlockSpec((1, HID), lambda i: (0, 0)),
      ],
      out_specs=pl.BlockSpec((NUM_SC, BLKR, 128), lambda i: (0, i, 0)),
      out_shape=jax.ShapeDtypeStruct((NUM_SC, NR, 128), jnp.float32),
  )(eps.reshape(1), h4, agg4, w1, b1, w2, b2)


def _pool_body(h_ref, batch_ref, wc_ref, bc_ref, out_ref, sums_ref, cnts_ref):
  i = pl.program_id(0)

  @pl.when(i == 0)
  def _():
    sums_ref[...] = jnp.zeros_like(sums_ref)
    cnts_ref[...] = jnp.zeros_like(cnts_ref)

  for q in range(NQ):
    b = batch_ref[0, q, :]
    oh = (b[:, None] == lax.broadcasted_iota(jnp.int32, (BLKR, NUM_GRAPHS), 1)
          ).astype(jnp.float32)
    hq = jnp.concatenate([h_ref[0][:, 32 * q:32 * q + 32],
                          h_ref[1][:, 32 * q:32 * q + 32]], axis=1)
    sums_ref[...] += lax.dot_general(oh, hq, (((0,), (0,)), ((), ())),
                                     preferred_element_type=jnp.float32)
    cnts_ref[...] += lax.dot_general(oh, oh, (((0,), (0,)), ((), ())),
                                     preferred_element_type=jnp.float32)

  @pl.when(i == pl.num_programs(0) - 1)
  def _():
    cb = jnp.dot(cnts_ref[...], jnp.ones((NUM_GRAPHS, NUM_GRAPHS),
                                         jnp.float32),
                 preferred_element_type=jnp.float32)
    pooled = sums_ref[...] / jnp.maximum(cb, 1.0)
    out_ref[...] = jnp.dot(pooled, wc_ref[...],
                           preferred_element_type=jnp.float32) + bc_ref[...]


def _pool(h4, batch4, wc, bc):
  return pl.pallas_call(
      _pool_body,
      grid=(NGRID,),
      in_specs=[
          pl.BlockSpec((NUM_SC, BLKR, 128), lambda i: (0, i, 0)),
          pl.BlockSpec((1, NQ, BLKR), lambda i: (i, 0, 0)),
          pl.BlockSpec((HID, NUM_CLASSES), lambda i: (0, 0)),
          pl.BlockSpec((1, NUM_CLASSES), lambda i: (0, 0)),
      ],
      out_specs=pl.BlockSpec((NUM_GRAPHS, NUM_CLASSES), lambda i: (0, 0)),
      out_shape=jax.ShapeDtypeStruct((NUM_GRAPHS, NUM_CLASSES), jnp.float32),
      scratch_shapes=[
          pltpu.VMEM((NUM_GRAPHS, HID), jnp.float32),
          pltpu.VMEM((NUM_GRAPHS, NUM_GRAPHS), jnp.float32),
      ],
  )(h4, batch4, wc, bc)


def kernel(x, edge_index, batch, params):
  src_p = (edge_index[0] % NR) * NQ + edge_index[0] // NR
  dst_p = (edge_index[1] % NR) * NQ + edge_index[1] // NR
  idx_all = jnp.stack([src_p.reshape(NCHUNK, NSUB, SUB),
                       dst_p.reshape(NCHUNK, NSUB, SUB)], axis=1)

  x64 = jnp.pad(x, ((0, NPAD - N), (0, HID - IN_DIM)))
  h4 = x64.reshape(NQ, NR, NUM_SC, HID // 2).transpose(2, 1, 0, 3).reshape(
      NUM_SC, NR, 128)
  w1_0 = jnp.pad(params["W1"][0], ((0, HID - IN_DIM), (0, 0)))
  batch4 = jnp.pad(batch, (0, NPAD - N), constant_values=NUM_GRAPHS).reshape(
      NQ, NGRID, BLKR).transpose(1, 0, 2)

  for layer in range(NUM_LAYER):
    h3 = h4.reshape(NUM_SC, NPAD, HID // 2)
    agg4 = _seg_sum(HID // 2)(h3, idx_all).reshape(NUM_SC, NR, 128)
    w1 = w1_0 if layer == 0 else params["W1"][layer]
    h4 = _mlp4(params["eps"][layer], h4, agg4, w1,
               params["b1"][layer].reshape(1, HID),
               params["W2"][layer], params["b2"][layer].reshape(1, HID))
  return _pool(h4, batch4, params["Wc"], params["bc"].reshape(1, NUM_CLASSES))
